# Initial kernel scaffold; baseline (speedup 1.0000x reference)
import jax
import jax.numpy as jnp
from jax.experimental import pallas as pl


def kernel(x, params, edge_src, edge_dst):
    raise NotImplementedError("write your pallas kernel here")


# speedup vs baseline: 1.2025x; 1.2025x over previous
import functools

import numpy as np
import jax
import jax.numpy as jnp
from jax import lax
from jax.experimental import pallas as pl
from jax.experimental.pallas import tpu as pltpu
from jax.experimental.pallas import tpu_sc as plsc

N_NODES = 10000
POS_CH = 16
EMB = 2 * POS_CH * 3
KIN = 2 * EMB
HID = 128
NUM_LAYERS = 4
SC_WORKERS = 32
SC_CHUNK = 128
BE = 512
BN = 1000


def _round_up(v, m):
    return (v + m - 1) // m * m


def _gelu(v):
    return 0.5 * v * (1.0 + lax.erf(v * np.float32(1.0 / np.sqrt(2.0))))


def _sc_gather(table, idx):
    b_total = idx.shape[0]
    d = table.shape[1]
    b_per_w = b_total // SC_WORKERS
    chunks = b_per_w // SC_CHUNK
    mesh = plsc.VectorSubcoreMesh(core_axis_name="c", subcore_axis_name="s")

    @functools.partial(
        pl.kernel,
        out_type=jax.ShapeDtypeStruct((b_total, d), table.dtype),
        mesh=mesh,
        scratch_types=[
            pltpu.VMEM((SC_CHUNK,), jnp.int32),
            pltpu.VMEM((SC_CHUNK, d), table.dtype),
            pltpu.SemaphoreType.DMA,
        ],
    )
    def gather_kernel(table_hbm, idx_hbm, out_hbm, idx_v, rows_v, sem):
        wid = lax.axis_index("s") * 2 + lax.axis_index("c")
        base = wid * b_per_w

        @pl.loop(0, chunks)
        def _(j):
            off = base + j * SC_CHUNK
            pltpu.sync_copy(idx_hbm.at[pl.ds(off, SC_CHUNK)], idx_v)
            pltpu.async_copy(table_hbm.at[idx_v], rows_v, sem).wait()
            pltpu.sync_copy(rows_v, out_hbm.at[pl.ds(off, SC_CHUNK)])

    return gather_kernel(table, idx)


def _mlp2_body(x_ref, w1_ref, b1_ref, w2_ref, b2_ref, o_ref):
    t = _gelu(jnp.dot(x_ref[...], w1_ref[...],
                      preferred_element_type=jnp.float32) + b1_ref[...])
    o_ref[...] = jnp.dot(t, w2_ref[...],
                         preferred_element_type=jnp.float32) + b2_ref[...]


def _enc_body(x_ref, w1_ref, b1_ref, w2_ref, b2_ref, fr_ref, ph_ref,
              h_ref, pe_ref):
    x = x_ref[...]
    t = _gelu(jnp.dot(x, w1_ref[...],
                      preferred_element_type=jnp.float32) + b1_ref[...])
    h_ref[...] = jnp.dot(t, w2_ref[...],
                         preferred_element_type=jnp.float32) + b2_ref[...]
    cols = [jnp.broadcast_to(x[:, c:c + 1], (x.shape[0], 32))
            for c in range(3)]
    pos_big = jnp.concatenate(cols, axis=1)
    g = jnp.sin(pos_big * fr_ref[...] + ph_ref[...])
    pe_ref[...] = jnp.pad(g, ((0, 0), (0, 32)))


def _edge_body(n_edges, ped_ref, pes_ref, hd_ref,
               w0d_ref, w0s_ref, b0_ref, w1_ref, b1_ref, w2_ref, b2_ref,
               c_ref, carry_ref):
    i = pl.program_id(0)

    @pl.when(i == 0)
    def _():
        carry_ref[...] = jnp.zeros_like(carry_ref)

    t = (jnp.dot(ped_ref[...], w0d_ref[...],
                 preferred_element_type=jnp.float32)
         + jnp.dot(pes_ref[...], w0s_ref[...],
                   preferred_element_type=jnp.float32)
         + b0_ref[...])
    t = _gelu(t)
    t = _gelu(jnp.dot(t, w1_ref[...],
                      preferred_element_type=jnp.float32) + b1_ref[...])
    k = jnp.dot(t, w2_ref[...],
                preferred_element_type=jnp.float32) + b2_ref[...]

    msg = k * hd_ref[...]
    rows = i * BE + lax.broadcasted_iota(jnp.int32, (BE, 1), 0)
    msg = jnp.where(rows < n_edges, msg, 0.0)

    s = 1
    while s < BE:
        msg = msg + jnp.pad(msg, ((s, 0), (0, 0)))[:BE]
        s *= 2
    c = msg + carry_ref[0:1, :]
    c_ref[...] = c
    carry_ref[0:1, :] = c[BE - 1:BE, :]


def _update_body(h_ref, ge_ref, gs_ref, oab_ref, gam_ref, bet_ref, o_ref):
    i = pl.program_id(0)
    ge = ge_ref[...]
    gs = gs_ref[...]
    rows = i * BN + lax.broadcasted_iota(jnp.int32, (BN, 1), 0)
    gs = jnp.where(rows == 0, 0.0, gs)
    cnt = (oab_ref[:, 1:2] - oab_ref[:, 0:1]).astype(jnp.float32)
    inv = 1.0 / jnp.maximum(cnt, 1.0)
    hn = h_ref[...] + (ge - gs) * inv
    mu = jnp.mean(hn, axis=1, keepdims=True)
    var = jnp.mean((hn - mu) ** 2, axis=1, keepdims=True)
    o_ref[...] = ((hn - mu) * lax.rsqrt(var + 1e-5) * gam_ref[...]
                  + bet_ref[...])


def _full(shape):
    return pl.BlockSpec(shape, lambda i: (0, 0))


def kernel(x, params, edge_src, edge_dst):
    n_edges = edge_src.shape[0]
    ep = _round_up(n_edges, SC_WORKERS * SC_CHUNK)
    dstp = jnp.pad(edge_dst.astype(jnp.int32), (0, ep - n_edges))
    srcp = jnp.pad(edge_src.astype(jnp.int32), (0, ep - n_edges))

    offsets = jnp.searchsorted(edge_src, jnp.arange(N_NODES + 1,
                                                    dtype=edge_src.dtype))
    offsets = offsets.astype(jnp.int32)
    idx_end = offsets[1:] - 1
    idx_start = jnp.maximum(offsets[:-1] - 1, 0)
    np2 = _round_up(2 * N_NODES, SC_WORKERS * SC_CHUNK)
    gidx = jnp.pad(jnp.concatenate([idx_end, idx_start]),
                   (0, np2 - 2 * N_NODES))
    oab = jnp.pad(jnp.stack([offsets[:-1], offsets[1:]], axis=1),
                  ((0, 0), (0, 6)))

    freqs = 1.0 / (10000.0 ** (np.arange(POS_CH, dtype=np.float32) / POS_CH))
    fr_half = np.concatenate([freqs, freqs])
    ph_half = np.concatenate([np.zeros(POS_CH, np.float32),
                              np.full(POS_CH, np.pi / 2, np.float32)])
    fr96 = jnp.asarray(np.tile(fr_half, 3)[None, :])
    ph96 = jnp.asarray(np.tile(ph_half, 3)[None, :])

    def row(v):
        return v.reshape(1, -1)

    h, pe = pl.pallas_call(
        _enc_body,
        grid=(N_NODES // BN,),
        in_specs=[
            pl.BlockSpec((BN, HID), lambda i: (i, 0)),
            _full((HID, HID)), _full((1, HID)),
            _full((HID, HID)), _full((1, HID)),
            _full((1, EMB)), _full((1, EMB)),
        ],
        out_specs=[pl.BlockSpec((BN, HID), lambda i: (i, 0)),
                   pl.BlockSpec((BN, HID), lambda i: (i, 0))],
        out_shape=[jax.ShapeDtypeStruct((N_NODES, HID), jnp.float32),
                   jax.ShapeDtypeStruct((N_NODES, HID), jnp.float32)],
    )(x, params['enc_w1'], row(params['enc_b1']),
      params['enc_w2'], row(params['enc_b2']), fr96, ph96)

    peg = _sc_gather(pe, jnp.concatenate([dstp, srcp]))

    ne_blocks = ep // BE
    for l in range(NUM_LAYERS):
        hd = _sc_gather(h, dstp)

        w0 = params[f'k{l}_w0']
        w0d = jnp.pad(w0[:EMB], ((0, HID - EMB), (0, 0)))
        w0s = jnp.pad(w0[EMB:], ((0, HID - EMB), (0, 0)))

        c = pl.pallas_call(
            functools.partial(_edge_body, n_edges),
            grid=(ne_blocks,),
            in_specs=[
                pl.BlockSpec((BE, HID), lambda i: (i, 0)),
                pl.BlockSpec((BE, HID), lambda i: (i + ne_blocks, 0)),
                pl.BlockSpec((BE, HID), lambda i: (i, 0)),
                _full((HID, HID)), _full((HID, HID)), _full((1, HID)),
                _full((HID, 2 * HID)), _full((1, 2 * HID)),
                _full((2 * HID, HID)), _full((1, HID)),
            ],
            out_specs=pl.BlockSpec((BE, HID), lambda i: (i, 0)),
            out_shape=jax.ShapeDtypeStruct((ep, HID), jnp.float32),
            scratch_shapes=[pltpu.VMEM((8, HID), jnp.float32)],
        )(peg, peg, hd,
          w0d, w0s, row(params[f'k{l}_b0']),
          params[f'k{l}_w1'], row(params[f'k{l}_b1']),
          params[f'k{l}_w2'], row(params[f'k{l}_b2']))

        gar = _sc_gather(c, gidx)

        h = pl.pallas_call(
            _update_body,
            grid=(N_NODES // BN,),
            in_specs=[
                pl.BlockSpec((BN, HID), lambda i: (i, 0)),
                pl.BlockSpec((BN, HID), lambda i: (i, 0)),
                pl.BlockSpec((BN, HID),
                             lambda i: (i + N_NODES // BN, 0)),
                pl.BlockSpec((BN, 8), lambda i: (i, 0)),
                _full((1, HID)), _full((1, HID)),
            ],
            out_specs=pl.BlockSpec((BN, HID), lambda i: (i, 0)),
            out_shape=jax.ShapeDtypeStruct((N_NODES, HID), jnp.float32),
        )(h, gar, gar, oab, row(params[f'ln{l}_g']), row(params[f'ln{l}_b']))

    hw2 = jnp.pad(params['head_w2'], ((0, 0), (0, 5)))
    hb2 = jnp.pad(params['head_b2'], (0, 5))
    out = pl.pallas_call(
        _mlp2_body,
        grid=(N_NODES // BN,),
        in_specs=[
            pl.BlockSpec((BN, HID), lambda i: (i, 0)),
            _full((HID, HID)), _full((1, HID)),
            _full((HID, 8)), _full((1, 8)),
        ],
        out_specs=pl.BlockSpec((BN, 8), lambda i: (i, 0)),
        out_shape=jax.ShapeDtypeStruct((N_NODES, 8), jnp.float32),
    )(h, params['head_w1'], row(params['head_b1']), hw2, row(hb2))

    return out[:, :3]


# speedup vs baseline: 1.3066x; 1.0865x over previous
import functools

import numpy as np
import jax
import jax.numpy as jnp
from jax import lax
from jax.experimental import pallas as pl
from jax.experimental.pallas import tpu as pltpu
from jax.experimental.pallas import tpu_sc as plsc

N_NODES = 10000
POS_CH = 16
EMB = 2 * POS_CH * 3
KIN = 2 * EMB
HID = 128
NUM_LAYERS = 4
SC_WORKERS = 32
SC_CHUNK = 128
BE = 512
BN = 1000


def _round_up(v, m):
    return (v + m - 1) // m * m


def _gelu(v):
    return 0.5 * v * (1.0 + lax.erf(v * np.float32(1.0 / np.sqrt(2.0))))


def _sc_gather(table, idx):
    b_total = idx.shape[0]
    d = table.shape[1]
    b_per_w = b_total // SC_WORKERS
    chunks = b_per_w // SC_CHUNK
    mesh = plsc.VectorSubcoreMesh(core_axis_name="c", subcore_axis_name="s")

    @functools.partial(
        pl.kernel,
        out_type=jax.ShapeDtypeStruct((b_total, d), table.dtype),
        mesh=mesh,
        scratch_types=[
            pltpu.VMEM((b_per_w,), jnp.int32),
            pltpu.VMEM((SC_CHUNK, d), table.dtype),
            pltpu.VMEM((SC_CHUNK, d), table.dtype),
            pltpu.SemaphoreType.DMA,
            pltpu.SemaphoreType.DMA,
        ],
    )
    def gather_kernel(table_hbm, idx_hbm, out_hbm, idx_v, buf0, buf1, s0, s1):
        wid = lax.axis_index("s") * 2 + lax.axis_index("c")
        base = wid * b_per_w
        pltpu.sync_copy(idx_hbm.at[pl.ds(base, b_per_w)], idx_v)
        bufs, sems = (buf0, buf1), (s0, s1)
        cp = pltpu.async_copy(
            table_hbm.at[idx_v.at[pl.ds(0, SC_CHUNK)]], bufs[0], sems[0])
        for j in range(chunks):
            nxt = None
            if j + 1 < chunks:
                nxt = pltpu.async_copy(
                    table_hbm.at[idx_v.at[pl.ds((j + 1) * SC_CHUNK, SC_CHUNK)]],
                    bufs[(j + 1) % 2], sems[(j + 1) % 2])
            cp.wait()
            pltpu.sync_copy(bufs[j % 2],
                            out_hbm.at[pl.ds(base + j * SC_CHUNK, SC_CHUNK)])
            cp = nxt

    return gather_kernel(table, idx)


def _mlp2_body(x_ref, w1_ref, b1_ref, w2_ref, b2_ref, o_ref):
    t = _gelu(jnp.dot(x_ref[...], w1_ref[...],
                      preferred_element_type=jnp.float32) + b1_ref[...])
    o_ref[...] = jnp.dot(t, w2_ref[...],
                         preferred_element_type=jnp.float32) + b2_ref[...]


def _enc_body(x_ref, w1_ref, b1_ref, w2_ref, b2_ref, fr_ref, ph_ref,
              h_ref, pe_ref):
    x = x_ref[...]
    t = _gelu(jnp.dot(x, w1_ref[...],
                      preferred_element_type=jnp.float32) + b1_ref[...])
    h_ref[...] = jnp.dot(t, w2_ref[...],
                         preferred_element_type=jnp.float32) + b2_ref[...]
    cols = [jnp.broadcast_to(x[:, c:c + 1], (x.shape[0], 32))
            for c in range(3)]
    pos_big = jnp.concatenate(cols, axis=1)
    g = jnp.sin(pos_big * fr_ref[...] + ph_ref[...])
    pe_ref[...] = jnp.pad(g, ((0, 0), (0, 32)))


NR = BE + 8


def _edge_body(n_edges, n0s_ref, ped_ref, pes_ref, hd_ref, src_ref,
               w0d_ref, w0s_ref, b0_ref, w1_ref, b1_ref, w2_ref, b2_ref,
               acc_ref):
    i = pl.program_id(0)

    @pl.when(i == 0)
    def _():
        acc_ref[...] = jnp.zeros_like(acc_ref)

    t = (jnp.dot(ped_ref[...], w0d_ref[...],
                 preferred_element_type=jnp.float32)
         + jnp.dot(pes_ref[...], w0s_ref[...],
                   preferred_element_type=jnp.float32)
         + b0_ref[...])
    t = _gelu(t)
    t = _gelu(jnp.dot(t, w1_ref[...],
                      preferred_element_type=jnp.float32) + b1_ref[...])
    k = jnp.dot(t, w2_ref[...],
                preferred_element_type=jnp.float32) + b2_ref[...]

    msg = k * hd_ref[...]
    rows = i * BE + lax.broadcasted_iota(jnp.int32, (BE, 1), 0)
    msg = jnp.where(rows < n_edges, msg, 0.0)

    n0 = n0s_ref[i]
    src_rel = src_ref[0, 0, :].reshape(1, BE) - n0
    oh = (lax.broadcasted_iota(jnp.int32, (NR, BE), 0)
          == src_rel).astype(jnp.float32)
    upd = jnp.dot(oh, msg, preferred_element_type=jnp.float32)
    acc_ref[pl.ds(n0, NR), :] += upd


def _update_body(h_ref, seg_ref, oab_ref, gam_ref, bet_ref, o_ref):
    cnt = (oab_ref[:, 1:2] - oab_ref[:, 0:1]).astype(jnp.float32)
    inv = 1.0 / jnp.maximum(cnt, 1.0)
    hn = h_ref[...] + seg_ref[...] * inv
    mu = jnp.mean(hn, axis=1, keepdims=True)
    var = jnp.mean((hn - mu) ** 2, axis=1, keepdims=True)
    o_ref[...] = ((hn - mu) * lax.rsqrt(var + 1e-5) * gam_ref[...]
                  + bet_ref[...])


def _full(shape):
    return pl.BlockSpec(shape, lambda i: (0, 0))


def kernel(x, params, edge_src, edge_dst):
    n_edges = edge_src.shape[0]
    ep = _round_up(n_edges, SC_WORKERS * SC_CHUNK)
    dstp = jnp.pad(edge_dst.astype(jnp.int32), (0, ep - n_edges))
    srcp = jnp.pad(edge_src.astype(jnp.int32), (0, ep - n_edges))

    offsets = jnp.searchsorted(edge_src, jnp.arange(N_NODES + 1,
                                                    dtype=edge_src.dtype))
    offsets = offsets.astype(jnp.int32)
    oab = jnp.pad(jnp.stack([offsets[:-1], offsets[1:]], axis=1),
                  ((0, 0), (0, 6)))
    n0s = (srcp[::BE] // 8) * 8
    srcb = srcp.reshape(-1, 1, BE)

    freqs = 1.0 / (10000.0 ** (np.arange(POS_CH, dtype=np.float32) / POS_CH))
    fr_half = np.concatenate([freqs, freqs])
    ph_half = np.concatenate([np.zeros(POS_CH, np.float32),
                              np.full(POS_CH, np.pi / 2, np.float32)])
    fr96 = jnp.asarray(np.tile(fr_half, 3)[None, :])
    ph96 = jnp.asarray(np.tile(ph_half, 3)[None, :])

    def row(v):
        return v.reshape(1, -1)

    h, pe = pl.pallas_call(
        _enc_body,
        grid=(N_NODES // BN,),
        in_specs=[
            pl.BlockSpec((BN, HID), lambda i: (i, 0)),
            _full((HID, HID)), _full((1, HID)),
            _full((HID, HID)), _full((1, HID)),
            _full((1, EMB)), _full((1, EMB)),
        ],
        out_specs=[pl.BlockSpec((BN, HID), lambda i: (i, 0)),
                   pl.BlockSpec((BN, HID), lambda i: (i, 0))],
        out_shape=[jax.ShapeDtypeStruct((N_NODES, HID), jnp.float32),
                   jax.ShapeDtypeStruct((N_NODES, HID), jnp.float32)],
    )(x, params['enc_w1'], row(params['enc_b1']),
      params['enc_w2'], row(params['enc_b2']), fr96, ph96)

    peg = _sc_gather(pe, jnp.concatenate([dstp, srcp]))

    ne_blocks = ep // BE
    n_pad = _round_up(N_NODES + NR, 8)
    for l in range(NUM_LAYERS):
        hd = _sc_gather(h, dstp)

        w0 = params[f'k{l}_w0']
        w0d = jnp.pad(w0[:EMB], ((0, HID - EMB), (0, 0)))
        w0s = jnp.pad(w0[EMB:], ((0, HID - EMB), (0, 0)))

        seg = pl.pallas_call(
            functools.partial(_edge_body, n_edges),
            grid=(ne_blocks,),
            in_specs=[
                pl.BlockSpec(memory_space=pltpu.SMEM),
                pl.BlockSpec((BE, HID), lambda i: (i, 0)),
                pl.BlockSpec((BE, HID), lambda i: (i + ne_blocks, 0)),
                pl.BlockSpec((BE, HID), lambda i: (i, 0)),
                pl.BlockSpec((1, 1, BE), lambda i: (i, 0, 0)),
                _full((HID, HID)), _full((HID, HID)), _full((1, HID)),
                _full((HID, 2 * HID)), _full((1, 2 * HID)),
                _full((2 * HID, HID)), _full((1, HID)),
            ],
            out_specs=pl.BlockSpec((n_pad, HID), lambda i: (0, 0)),
            out_shape=jax.ShapeDtypeStruct((n_pad, HID), jnp.float32),
        )(n0s, peg, peg, hd, srcb,
          w0d, w0s, row(params[f'k{l}_b0']),
          params[f'k{l}_w1'], row(params[f'k{l}_b1']),
          params[f'k{l}_w2'], row(params[f'k{l}_b2']))

        h = pl.pallas_call(
            _update_body,
            grid=(N_NODES // BN,),
            in_specs=[
                pl.BlockSpec((BN, HID), lambda i: (i, 0)),
                pl.BlockSpec((BN, HID), lambda i: (i, 0)),
                pl.BlockSpec((BN, 8), lambda i: (i, 0)),
                _full((1, HID)), _full((1, HID)),
            ],
            out_specs=pl.BlockSpec((BN, HID), lambda i: (i, 0)),
            out_shape=jax.ShapeDtypeStruct((N_NODES, HID), jnp.float32),
        )(h, seg, oab, row(params[f'ln{l}_g']), row(params[f'ln{l}_b']))

    hw2 = jnp.pad(params['head_w2'], ((0, 0), (0, 5)))
    hb2 = jnp.pad(params['head_b2'], (0, 5))
    out = pl.pallas_call(
        _mlp2_body,
        grid=(N_NODES // BN,),
        in_specs=[
            pl.BlockSpec((BN, HID), lambda i: (i, 0)),
            _full((HID, HID)), _full((1, HID)),
            _full((HID, 8)), _full((1, 8)),
        ],
        out_specs=pl.BlockSpec((BN, 8), lambda i: (i, 0)),
        out_shape=jax.ShapeDtypeStruct((N_NODES, 8), jnp.float32),
    )(h, params['head_w1'], row(params['head_b1']), hw2, row(hb2))

    return out[:, :3]


# speedup vs baseline: 1.6948x; 1.2971x over previous
import functools

import numpy as np
import jax
import jax.numpy as jnp
from jax import lax
from jax.experimental import pallas as pl
from jax.experimental.pallas import tpu as pltpu
from jax.experimental.pallas import tpu_sc as plsc

N_NODES = 10000
POS_CH = 16
EMB = 2 * POS_CH * 3
KIN = 2 * EMB
HID = 128
NUM_LAYERS = 4
SC_WORKERS = 32
SC_CHUNK = 128
BE = 512
BN = 1000


def _round_up(v, m):
    return (v + m - 1) // m * m


def _gelu(v):
    return 0.5 * v * (1.0 + lax.erf(v * np.float32(1.0 / np.sqrt(2.0))))


def _sc_gather(table, idx):
    b_total = idx.shape[0]
    d = table.shape[1]
    b_per_w = b_total // SC_WORKERS
    chunks = b_per_w // SC_CHUNK
    mesh = plsc.VectorSubcoreMesh(core_axis_name="c", subcore_axis_name="s")

    nbuf = min(4, chunks)

    @functools.partial(
        pl.kernel,
        out_type=jax.ShapeDtypeStruct((b_total, d), table.dtype),
        mesh=mesh,
        scratch_types=(
            [pltpu.VMEM((b_per_w,), jnp.int32)]
            + [pltpu.VMEM((SC_CHUNK, d), table.dtype)] * nbuf
            + [pltpu.SemaphoreType.DMA] * (2 * nbuf)
        ),
    )
    def gather_kernel(table_hbm, idx_hbm, out_hbm, idx_v, *rest):
        bufs = rest[:nbuf]
        gsem = rest[nbuf:2 * nbuf]
        wsem = rest[2 * nbuf:]
        wid = lax.axis_index("s") * 2 + lax.axis_index("c")
        base = wid * b_per_w
        pltpu.sync_copy(idx_hbm.at[pl.ds(base, b_per_w)], idx_v)

        def gather_chunk(j):
            return pltpu.async_copy(
                table_hbm.at[idx_v.at[pl.ds(j * SC_CHUNK, SC_CHUNK)]],
                bufs[j % nbuf], gsem[j % nbuf])

        cps = [None] * nbuf
        wbs = [None] * nbuf
        for j in range(nbuf - 1):
            cps[j % nbuf] = gather_chunk(j)
        for j in range(chunks):
            cps[j % nbuf].wait()
            wbs[j % nbuf] = pltpu.async_copy(
                bufs[j % nbuf],
                out_hbm.at[pl.ds(base + j * SC_CHUNK, SC_CHUNK)],
                wsem[j % nbuf])
            nj = j + nbuf - 1
            if nj < chunks:
                if wbs[nj % nbuf] is not None:
                    wbs[nj % nbuf].wait()
                    wbs[nj % nbuf] = None
                cps[nj % nbuf] = gather_chunk(nj)
        for wb in wbs:
            if wb is not None:
                wb.wait()

    return gather_kernel(table, idx)


def _mlp2_body(x_ref, w1_ref, b1_ref, w2_ref, b2_ref, o_ref):
    t = _gelu(jnp.dot(x_ref[...], w1_ref[...],
                      preferred_element_type=jnp.float32) + b1_ref[...])
    o_ref[...] = jnp.dot(t, w2_ref[...],
                         preferred_element_type=jnp.float32) + b2_ref[...]


def _enc_body(x_ref, w1_ref, b1_ref, w2_ref, b2_ref, fr_ref, ph_ref,
              h_ref, pe_ref):
    x = x_ref[...]
    t = _gelu(jnp.dot(x, w1_ref[...],
                      preferred_element_type=jnp.float32) + b1_ref[...])
    h_ref[...] = jnp.dot(t, w2_ref[...],
                         preferred_element_type=jnp.float32) + b2_ref[...]
    cols = [jnp.broadcast_to(x[:, c:c + 1], (x.shape[0], 32))
            for c in range(3)]
    pos_big = jnp.concatenate(cols, axis=1)
    g = jnp.sin(pos_big * fr_ref[...] + ph_ref[...])
    pe_ref[...] = jnp.pad(g, ((0, 0), (0, 32)))


NR = BE + 8


def _edge_body(n_edges, with_cnt, n0s_ref, ped_ref, pes_ref, hd_ref, src_ref,
               w0d_ref, w0s_ref, b0_ref, w1_ref, b1_ref, w2_ref, b2_ref,
               acc_ref, *cnt_out):
    i = pl.program_id(0)

    @pl.when(i == 0)
    def _():
        acc_ref[...] = jnp.zeros_like(acc_ref)
        if with_cnt:
            cnt_out[0][...] = jnp.zeros_like(cnt_out[0])

    t = (jnp.dot(ped_ref[...], w0d_ref[...],
                 preferred_element_type=jnp.float32)
         + jnp.dot(pes_ref[...], w0s_ref[...],
                   preferred_element_type=jnp.float32)
         + b0_ref[...])
    t = _gelu(t)
    t = _gelu(jnp.dot(t, w1_ref[...],
                      preferred_element_type=jnp.float32) + b1_ref[...])
    k = jnp.dot(t, w2_ref[...],
                preferred_element_type=jnp.float32) + b2_ref[...]

    msg = k * hd_ref[...]
    rows = i * BE + lax.broadcasted_iota(jnp.int32, (BE, 1), 0)
    msg = jnp.where(rows < n_edges, msg, 0.0)

    n0 = n0s_ref[i]
    src_rel = src_ref[0, 0, :].reshape(1, BE) - n0
    oh = (lax.broadcasted_iota(jnp.int32, (NR, BE), 0)
          == src_rel).astype(jnp.float32)
    upd = jnp.dot(oh, msg, preferred_element_type=jnp.float32)
    acc_ref[pl.ds(n0, NR), :] += upd
    if with_cnt:
        onesm = jnp.where(rows < n_edges,
                          jnp.float32(1.0), jnp.float32(0.0))
        cnt_out[0][pl.ds(n0, NR), :] += jnp.dot(
            oh, jnp.broadcast_to(onesm, (BE, 8)),
            preferred_element_type=jnp.float32)


def _update_body(h_ref, seg_ref, cnt_ref, gam_ref, bet_ref, o_ref):
    inv = 1.0 / jnp.maximum(cnt_ref[:, 0:1], 1.0)
    hn = h_ref[...] + seg_ref[...] * inv
    mu = jnp.mean(hn, axis=1, keepdims=True)
    var = jnp.mean((hn - mu) ** 2, axis=1, keepdims=True)
    o_ref[...] = ((hn - mu) * lax.rsqrt(var + 1e-5) * gam_ref[...]
                  + bet_ref[...])


def _full(shape):
    return pl.BlockSpec(shape, lambda i: (0, 0))


def kernel(x, params, edge_src, edge_dst):
    n_edges = edge_src.shape[0]
    ep = _round_up(n_edges, SC_WORKERS * SC_CHUNK)
    dstp = jnp.pad(edge_dst.astype(jnp.int32), (0, ep - n_edges))
    srcp = jnp.pad(edge_src.astype(jnp.int32), (0, ep - n_edges))

    n0s = (srcp[::BE] // 8) * 8
    srcb = srcp.reshape(-1, 1, BE)

    freqs = 1.0 / (10000.0 ** (np.arange(POS_CH, dtype=np.float32) / POS_CH))
    fr_half = np.concatenate([freqs, freqs])
    ph_half = np.concatenate([np.zeros(POS_CH, np.float32),
                              np.full(POS_CH, np.pi / 2, np.float32)])
    fr96 = jnp.asarray(np.tile(fr_half, 3)[None, :])
    ph96 = jnp.asarray(np.tile(ph_half, 3)[None, :])

    def row(v):
        return v.reshape(1, -1)

    h, pe = pl.pallas_call(
        _enc_body,
        grid=(N_NODES // BN,),
        in_specs=[
            pl.BlockSpec((BN, HID), lambda i: (i, 0)),
            _full((HID, HID)), _full((1, HID)),
            _full((HID, HID)), _full((1, HID)),
            _full((1, EMB)), _full((1, EMB)),
        ],
        out_specs=[pl.BlockSpec((BN, HID), lambda i: (i, 0)),
                   pl.BlockSpec((BN, HID), lambda i: (i, 0))],
        out_shape=[jax.ShapeDtypeStruct((N_NODES, HID), jnp.float32),
                   jax.ShapeDtypeStruct((N_NODES, HID), jnp.float32)],
    )(x, params['enc_w1'], row(params['enc_b1']),
      params['enc_w2'], row(params['enc_b2']), fr96, ph96)

    peg = _sc_gather(pe, jnp.concatenate([dstp, srcp]))

    ne_blocks = ep // BE
    n_pad = _round_up(N_NODES + NR, 8)
    cnt = None
    for l in range(NUM_LAYERS):
        hd = _sc_gather(h, dstp)

        w0 = params[f'k{l}_w0']
        w0d = jnp.pad(w0[:EMB], ((0, HID - EMB), (0, 0)))
        w0s = jnp.pad(w0[EMB:], ((0, HID - EMB), (0, 0)))

        with_cnt = l == 0
        out_specs = [pl.BlockSpec((n_pad, HID), lambda i: (0, 0))]
        out_shape = [jax.ShapeDtypeStruct((n_pad, HID), jnp.float32)]
        if with_cnt:
            out_specs.append(pl.BlockSpec((n_pad, 8), lambda i: (0, 0)))
            out_shape.append(jax.ShapeDtypeStruct((n_pad, 8), jnp.float32))
        res = pl.pallas_call(
            functools.partial(_edge_body, n_edges, with_cnt),
            grid=(ne_blocks,),
            in_specs=[
                pl.BlockSpec(memory_space=pltpu.SMEM),
                pl.BlockSpec((BE, HID), lambda i: (i, 0)),
                pl.BlockSpec((BE, HID), lambda i: (i + ne_blocks, 0)),
                pl.BlockSpec((BE, HID), lambda i: (i, 0)),
                pl.BlockSpec((1, 1, BE), lambda i: (i, 0, 0)),
                _full((HID, HID)), _full((HID, HID)), _full((1, HID)),
                _full((HID, 2 * HID)), _full((1, 2 * HID)),
                _full((2 * HID, HID)), _full((1, HID)),
            ],
            out_specs=out_specs,
            out_shape=out_shape,
        )(n0s, peg, peg, hd, srcb,
          w0d, w0s, row(params[f'k{l}_b0']),
          params[f'k{l}_w1'], row(params[f'k{l}_b1']),
          params[f'k{l}_w2'], row(params[f'k{l}_b2']))
        if with_cnt:
            seg, cnt = res
        else:
            seg, = res

        h = pl.pallas_call(
            _update_body,
            grid=(N_NODES // BN,),
            in_specs=[
                pl.BlockSpec((BN, HID), lambda i: (i, 0)),
                pl.BlockSpec((BN, HID), lambda i: (i, 0)),
                pl.BlockSpec((BN, 8), lambda i: (i, 0)),
                _full((1, HID)), _full((1, HID)),
            ],
            out_specs=pl.BlockSpec((BN, HID), lambda i: (i, 0)),
            out_shape=jax.ShapeDtypeStruct((N_NODES, HID), jnp.float32),
        )(h, seg, cnt, row(params[f'ln{l}_g']), row(params[f'ln{l}_b']))

    hw2 = jnp.pad(params['head_w2'], ((0, 0), (0, 5)))
    hb2 = jnp.pad(params['head_b2'], (0, 5))
    out = pl.pallas_call(
        _mlp2_body,
        grid=(N_NODES // BN,),
        in_specs=[
            pl.BlockSpec((BN, HID), lambda i: (i, 0)),
            _full((HID, HID)), _full((1, HID)),
            _full((HID, 8)), _full((1, 8)),
        ],
        out_specs=pl.BlockSpec((BN, 8), lambda i: (i, 0)),
        out_shape=jax.ShapeDtypeStruct((N_NODES, 8), jnp.float32),
    )(h, params['head_w1'], row(params['head_b1']), hw2, row(hb2))

    return out[:, :3]


# speedup vs baseline: 3.7192x; 2.1944x over previous
import functools

import numpy as np
import jax
import jax.numpy as jnp
from jax import lax
from jax.experimental import pallas as pl
from jax.experimental.pallas import tpu as pltpu
from jax.experimental.pallas import tpu_sc as plsc

N_NODES = 10000
POS_CH = 16
EMB = 2 * POS_CH * 3
KIN = 2 * EMB
HID = 128
NUM_LAYERS = 4
SC_WORKERS = 32
SC_CHUNK = 128
BE = 512
BN = 1000


def _round_up(v, m):
    return (v + m - 1) // m * m


def _gelu(v):
    return 0.5 * v * (1.0 + lax.erf(v * np.float32(1.0 / np.sqrt(2.0))))


def _sc_gather(table, idx):
    b_total = idx.shape[0]
    d = table.shape[1]
    b_per_w = b_total // SC_WORKERS
    chunks = b_per_w // SC_CHUNK
    mesh = plsc.VectorSubcoreMesh(core_axis_name="c", subcore_axis_name="s")

    nbuf = min(4, chunks)

    @functools.partial(
        pl.kernel,
        out_type=jax.ShapeDtypeStruct((b_total, d), table.dtype),
        mesh=mesh,
        scratch_types=(
            [pltpu.VMEM((b_per_w,), jnp.int32)]
            + [pltpu.VMEM((SC_CHUNK, d), table.dtype)] * nbuf
            + [pltpu.SemaphoreType.DMA] * (2 * nbuf)
        ),
    )
    def gather_kernel(table_hbm, idx_hbm, out_hbm, idx_v, *rest):
        bufs = rest[:nbuf]
        gsem = rest[nbuf:2 * nbuf]
        wsem = rest[2 * nbuf:]
        wid = lax.axis_index("s") * 2 + lax.axis_index("c")
        base = wid * b_per_w
        pltpu.sync_copy(idx_hbm.at[pl.ds(base, b_per_w)], idx_v)

        def gather_chunk(j):
            return pltpu.async_copy(
                table_hbm.at[idx_v.at[pl.ds(j * SC_CHUNK, SC_CHUNK)]],
                bufs[j % nbuf], gsem[j % nbuf])

        cps = [None] * nbuf
        wbs = [None] * nbuf
        for j in range(nbuf - 1):
            cps[j % nbuf] = gather_chunk(j)
        for j in range(chunks):
            cps[j % nbuf].wait()
            wbs[j % nbuf] = pltpu.async_copy(
                bufs[j % nbuf],
                out_hbm.at[pl.ds(base + j * SC_CHUNK, SC_CHUNK)],
                wsem[j % nbuf])
            nj = j + nbuf - 1
            if nj < chunks:
                if wbs[nj % nbuf] is not None:
                    wbs[nj % nbuf].wait()
                    wbs[nj % nbuf] = None
                cps[nj % nbuf] = gather_chunk(nj)
        for wb in wbs:
            if wb is not None:
                wb.wait()

    return gather_kernel(table, idx)


def _mlp2_body(x_ref, w1_ref, b1_ref, w2_ref, b2_ref, o_ref):
    t = _gelu(jnp.dot(x_ref[...], w1_ref[...],
                      preferred_element_type=jnp.float32) + b1_ref[...])
    o_ref[...] = jnp.dot(t, w2_ref[...],
                         preferred_element_type=jnp.float32) + b2_ref[...]


def _enc_body(x_ref, w1_ref, b1_ref, w2_ref, b2_ref, fr_ref, ph_ref,
              h_ref, pe_ref):
    x = x_ref[...]
    t = _gelu(jnp.dot(x, w1_ref[...],
                      preferred_element_type=jnp.float32) + b1_ref[...])
    h_ref[...] = jnp.dot(t, w2_ref[...],
                         preferred_element_type=jnp.float32) + b2_ref[...]
    cols = [jnp.broadcast_to(x[:, c:c + 1], (x.shape[0], 32))
            for c in range(3)]
    pos_big = jnp.concatenate(cols, axis=1)
    g = jnp.sin(pos_big * fr_ref[...] + ph_ref[...])
    pe_ref[...] = jnp.pad(g, ((0, 0), (0, 32)))


NR = BE + 8


def _edge_body(n_edges, with_cnt, n0s_ref, ped_ref, pes_ref, hd_ref, src_ref,
               w0d_ref, w0s_ref, b0_ref, w1_ref, b1_ref, w2_ref, b2_ref,
               acc_ref, *cnt_out):
    i = pl.program_id(0)

    @pl.when(i == 0)
    def _():
        acc_ref[...] = jnp.zeros_like(acc_ref)
        if with_cnt:
            cnt_out[0][...] = jnp.zeros_like(cnt_out[0])

    t = (jnp.dot(ped_ref[...].astype(jnp.float32), w0d_ref[...],
                 preferred_element_type=jnp.float32)
         + jnp.dot(pes_ref[...].astype(jnp.float32), w0s_ref[...],
                   preferred_element_type=jnp.float32)
         + b0_ref[...])
    t = _gelu(t)
    t = _gelu(jnp.dot(t, w1_ref[...],
                      preferred_element_type=jnp.float32) + b1_ref[...])
    k = jnp.dot(t, w2_ref[...],
                preferred_element_type=jnp.float32) + b2_ref[...]

    msg = k * hd_ref[...].astype(jnp.float32)
    rows = i * BE + lax.broadcasted_iota(jnp.int32, (BE, 1), 0)
    msg = jnp.where(rows < n_edges, msg, 0.0)

    n0 = n0s_ref[i]
    src_rel = src_ref[0, 0, :].reshape(1, BE) - n0
    oh = (lax.broadcasted_iota(jnp.int32, (NR, BE), 0)
          == src_rel).astype(jnp.float32)
    upd = jnp.dot(oh, msg, preferred_element_type=jnp.float32)
    acc_ref[pl.ds(n0, NR), :] += upd
    if with_cnt:
        onesm = jnp.where(rows < n_edges,
                          jnp.float32(1.0), jnp.float32(0.0))
        cnt_out[0][pl.ds(n0, NR), :] += jnp.dot(
            oh, jnp.broadcast_to(onesm, (BE, 8)),
            preferred_element_type=jnp.float32)


def _update_body(h_ref, seg_ref, cnt_ref, gam_ref, bet_ref, o_ref):
    inv = 1.0 / jnp.maximum(cnt_ref[:, 0:1], 1.0)
    hn = h_ref[...] + seg_ref[...] * inv
    mu = jnp.mean(hn, axis=1, keepdims=True)
    var = jnp.mean((hn - mu) ** 2, axis=1, keepdims=True)
    o_ref[...] = ((hn - mu) * lax.rsqrt(var + 1e-5) * gam_ref[...]
                  + bet_ref[...])


def _full(shape):
    return pl.BlockSpec(shape, lambda i: (0, 0))


def kernel(x, params, edge_src, edge_dst):
    n_edges = edge_src.shape[0]
    ep = _round_up(n_edges, SC_WORKERS * SC_CHUNK)
    spread = (jnp.arange(ep - n_edges, dtype=jnp.int32) * 7919) % N_NODES
    dstp = jnp.concatenate([edge_dst.astype(jnp.int32), spread])
    srcp = jnp.concatenate([edge_src.astype(jnp.int32), spread])

    n0s = (srcp[::BE] // 8) * 8
    srcb = srcp.reshape(-1, 1, BE)

    freqs = 1.0 / (10000.0 ** (np.arange(POS_CH, dtype=np.float32) / POS_CH))
    fr_half = np.concatenate([freqs, freqs])
    ph_half = np.concatenate([np.zeros(POS_CH, np.float32),
                              np.full(POS_CH, np.pi / 2, np.float32)])
    fr96 = jnp.asarray(np.tile(fr_half, 3)[None, :])
    ph96 = jnp.asarray(np.tile(ph_half, 3)[None, :])

    def row(v):
        return v.reshape(1, -1)

    h, pe = pl.pallas_call(
        _enc_body,
        grid=(N_NODES // BN,),
        in_specs=[
            pl.BlockSpec((BN, HID), lambda i: (i, 0)),
            _full((HID, HID)), _full((1, HID)),
            _full((HID, HID)), _full((1, HID)),
            _full((1, EMB)), _full((1, EMB)),
        ],
        out_specs=[pl.BlockSpec((BN, HID), lambda i: (i, 0)),
                   pl.BlockSpec((BN, HID), lambda i: (i, 0))],
        out_shape=[jax.ShapeDtypeStruct((N_NODES, HID), jnp.float32),
                   jax.ShapeDtypeStruct((N_NODES, HID), jnp.float32)],
    )(x, params['enc_w1'], row(params['enc_b1']),
      params['enc_w2'], row(params['enc_b2']), fr96, ph96)

    peg = _sc_gather(pe, jnp.concatenate([dstp, srcp]))

    ne_blocks = ep // BE
    n_pad = _round_up(N_NODES + NR, 8)
    cnt = None
    for l in range(NUM_LAYERS):
        hd = _sc_gather(h, dstp)

        w0 = params[f'k{l}_w0']
        w0d = jnp.pad(w0[:EMB], ((0, HID - EMB), (0, 0)))
        w0s = jnp.pad(w0[EMB:], ((0, HID - EMB), (0, 0)))

        with_cnt = l == 0
        out_specs = [pl.BlockSpec((n_pad, HID), lambda i: (0, 0))]
        out_shape = [jax.ShapeDtypeStruct((n_pad, HID), jnp.float32)]
        if with_cnt:
            out_specs.append(pl.BlockSpec((n_pad, 8), lambda i: (0, 0)))
            out_shape.append(jax.ShapeDtypeStruct((n_pad, 8), jnp.float32))
        res = pl.pallas_call(
            functools.partial(_edge_body, n_edges, with_cnt),
            grid=(ne_blocks,),
            in_specs=[
                pl.BlockSpec(memory_space=pltpu.SMEM),
                pl.BlockSpec((BE, HID), lambda i: (i, 0)),
                pl.BlockSpec((BE, HID), lambda i: (i + ne_blocks, 0)),
                pl.BlockSpec((BE, HID), lambda i: (i, 0)),
                pl.BlockSpec((1, 1, BE), lambda i: (i, 0, 0)),
                _full((HID, HID)), _full((HID, HID)), _full((1, HID)),
                _full((HID, 2 * HID)), _full((1, 2 * HID)),
                _full((2 * HID, HID)), _full((1, HID)),
            ],
            out_specs=out_specs,
            out_shape=out_shape,
        )(n0s, peg, peg, hd, srcb,
          w0d, w0s, row(params[f'k{l}_b0']),
          params[f'k{l}_w1'], row(params[f'k{l}_b1']),
          params[f'k{l}_w2'], row(params[f'k{l}_b2']))
        if with_cnt:
            seg, cnt = res
        else:
            seg, = res

        h = pl.pallas_call(
            _update_body,
            grid=(N_NODES // BN,),
            in_specs=[
                pl.BlockSpec((BN, HID), lambda i: (i, 0)),
                pl.BlockSpec((BN, HID), lambda i: (i, 0)),
                pl.BlockSpec((BN, 8), lambda i: (i, 0)),
                _full((1, HID)), _full((1, HID)),
            ],
            out_specs=pl.BlockSpec((BN, HID), lambda i: (i, 0)),
            out_shape=jax.ShapeDtypeStruct((N_NODES, HID), jnp.float32),
        )(h, seg, cnt, row(params[f'ln{l}_g']), row(params[f'ln{l}_b']))

    hw2 = jnp.pad(params['head_w2'], ((0, 0), (0, 5)))
    hb2 = jnp.pad(params['head_b2'], (0, 5))
    out = pl.pallas_call(
        _mlp2_body,
        grid=(N_NODES // BN,),
        in_specs=[
            pl.BlockSpec((BN, HID), lambda i: (i, 0)),
            _full((HID, HID)), _full((1, HID)),
            _full((HID, 8)), _full((1, 8)),
        ],
        out_specs=pl.BlockSpec((BN, 8), lambda i: (i, 0)),
        out_shape=jax.ShapeDtypeStruct((N_NODES, 8), jnp.float32),
    )(h, params['head_w1'], row(params['head_b1']), hw2, row(hb2))

    return out[:, :3]
